# baseline (speedup 1.0000x reference)
import jax
import jax.numpy as jnp
from jax import lax
from jax.experimental import pallas as pl
from jax.experimental.pallas import tpu as pltpu

_NUM_CODES = 1024
_DIM = 64
_ROWS = 16384
_R = 2048
_GRID = _ROWS // _R
_COMMIT = 0.25


def _vq_body(x_ref, embt_ref, emb_ref,
             enc_ref, q_ref, loss_ref, perp_ref,
             enorm_ref, counts_ref, acc_ref):
    step = pl.program_id(0)

    @pl.when(step == 0)
    def _init():
        embt0 = embt_ref[...]
        enorm_ref[...] = jnp.sum(embt0 * embt0, axis=0, keepdims=True)
        counts_ref[...] = jnp.zeros_like(counts_ref)
        acc_ref[0] = 0.0

    x = x_ref[...]
    xnorm = jnp.sum(x * x, axis=1, keepdims=True)
    xm2 = x * (-2.0)
    scores_m2 = lax.dot_general(xm2, embt_ref[...],
                                (((1,), (0,)), ((), ())),
                                preferred_element_type=jnp.float32)
    dist = (xnorm + enorm_ref[...]) + scores_m2
    minv = jnp.min(dist, axis=1, keepdims=True)
    iota = lax.broadcasted_iota(jnp.int32, (1, _NUM_CODES), 1).astype(jnp.float32)
    masked = jnp.where(dist == minv, iota, float(_NUM_CODES))
    idxf = jnp.min(masked, axis=1, keepdims=True)
    one_hot = jnp.where(masked == idxf, 1.0, 0.0)
    enc_ref[...] = one_hot
    q = lax.dot_general(one_hot, emb_ref[...],
                        (((1,), (0,)), ((), ())),
                        preferred_element_type=jnp.float32)
    qt0 = lax.transpose(q[:_R // 2, :], (1, 0))
    qt1 = lax.transpose(q[_R // 2:, :], (1, 0))
    q_ref[...] = jnp.concatenate([qt0, qt1], axis=0)
    d = q - x
    acc_ref[0] += jnp.sum(d * d)
    ones_row = jnp.ones((1, _R), jnp.float32)
    counts_ref[...] += lax.dot_general(ones_row, one_hot,
                                       (((1,), (0,)), ((), ())),
                                       preferred_element_type=jnp.float32)

    @pl.when(step == _GRID - 1)
    def _fin():
        mean_sq = acc_ref[0] / (_ROWS * _DIM)
        loss_ref[...] = jnp.full((1, 1), mean_sq + _COMMIT * mean_sq,
                                 jnp.float32)
        probs = counts_ref[...] / _ROWS
        ent = jnp.sum(probs * jnp.log(probs + 1e-10), keepdims=True)
        perp_ref[...] = jnp.exp(-ent).reshape(1, 1)


def kernel(inputs, embedding_weight):
    b, c, h, w = inputs.shape
    x = jnp.transpose(inputs, (0, 2, 3, 1)).reshape(_ROWS, _DIM)
    embt = embedding_weight.T
    enc, q, loss, perp = pl.pallas_call(
        _vq_body,
        grid=(_GRID,),
        in_specs=[
            pl.BlockSpec((_R, _DIM), lambda i: (i, 0)),
            pl.BlockSpec((_DIM, _NUM_CODES), lambda i: (0, 0)),
            pl.BlockSpec((_NUM_CODES, _DIM), lambda i: (0, 0)),
        ],
        out_specs=[
            pl.BlockSpec((_R, _NUM_CODES), lambda i: (i, 0)),
            pl.BlockSpec((2 * _DIM, _R // 2), lambda i: (i, 0)),
            pl.BlockSpec((1, 1), lambda i: (0, 0)),
            pl.BlockSpec((1, 1), lambda i: (0, 0)),
        ],
        out_shape=[
            jax.ShapeDtypeStruct((_ROWS, _NUM_CODES), jnp.float32),
            jax.ShapeDtypeStruct((_ROWS // (_R // 2) * _DIM, _R // 2), jnp.float32),
            jax.ShapeDtypeStruct((1, 1), jnp.float32),
            jax.ShapeDtypeStruct((1, 1), jnp.float32),
        ],
        scratch_shapes=[
            pltpu.VMEM((1, _NUM_CODES), jnp.float32),
            pltpu.VMEM((1, _NUM_CODES), jnp.float32),
            pltpu.SMEM((1,), jnp.float32),
        ],
    )(x, embt, embedding_weight)
    quantized_out = q.reshape(b, c, h, w)
    encodings_out = enc.reshape(b, h, w, _NUM_CODES)
    return quantized_out, loss.reshape(()), perp.reshape(()), encodings_out


# speedup vs baseline: 1.3380x; 1.3380x over previous
import jax
import jax.numpy as jnp
from jax import lax
from jax.experimental import pallas as pl
from jax.experimental.pallas import tpu as pltpu

_NUM_CODES = 1024
_DIM = 64
_ROWS = 16384
_R = 2048
_GRID = _ROWS // _R
_COMMIT = 0.25


def _vq_body(x_ref, embt_ref, emb_ref,
             enc_ref, q_ref, loss_ref, perp_ref,
             enorm_ref, counts_ref, acc_ref):
    step = pl.program_id(0)

    @pl.when(step == 0)
    def _init():
        embt0 = embt_ref[...]
        enorm_ref[...] = jnp.sum(embt0 * embt0, axis=0, keepdims=True)
        counts_ref[...] = jnp.zeros_like(counts_ref)
        acc_ref[0] = 0.0

    x = x_ref[...]
    xnorm = jnp.sum(x * x, axis=1, keepdims=True)
    xm2 = x * (-2.0)
    scores_m2 = lax.dot_general(xm2, embt_ref[...],
                                (((1,), (0,)), ((), ())),
                                preferred_element_type=jnp.float32)
    dist = (xnorm + enorm_ref[...]) + scores_m2
    minv = jnp.min(dist, axis=1, keepdims=True)
    iota = lax.broadcasted_iota(jnp.int32, (1, _NUM_CODES), 1).astype(jnp.float32)
    masked = jnp.where(dist == minv, iota, float(_NUM_CODES))
    idxf = jnp.min(masked, axis=1, keepdims=True)
    one_hot = jnp.where(masked == idxf, 1.0, 0.0)
    enc_ref[...] = one_hot
    q = lax.dot_general(one_hot, emb_ref[...],
                        (((1,), (0,)), ((), ())),
                        preferred_element_type=jnp.float32)
    q_ref[...] = q
    d = q - x
    acc_ref[0] += jnp.sum(d * d)
    ones_row = jnp.ones((1, _R), jnp.float32)
    counts_ref[...] += lax.dot_general(ones_row, one_hot,
                                       (((1,), (0,)), ((), ())),
                                       preferred_element_type=jnp.float32)

    @pl.when(step == _GRID - 1)
    def _fin():
        mean_sq = acc_ref[0] / (_ROWS * _DIM)
        loss_ref[...] = jnp.full((1, 1), mean_sq + _COMMIT * mean_sq,
                                 jnp.float32)
        probs = counts_ref[...] / _ROWS
        ent = jnp.sum(probs * jnp.log(probs + 1e-10), keepdims=True)
        perp_ref[...] = jnp.exp(-ent).reshape(1, 1)


def kernel(inputs, embedding_weight):
    b, c, h, w = inputs.shape
    x = jnp.transpose(inputs, (0, 2, 3, 1)).reshape(_ROWS, _DIM)
    embt = embedding_weight.T
    enc, q, loss, perp = pl.pallas_call(
        _vq_body,
        grid=(_GRID,),
        in_specs=[
            pl.BlockSpec((_R, _DIM), lambda i: (i, 0)),
            pl.BlockSpec((_DIM, _NUM_CODES), lambda i: (0, 0)),
            pl.BlockSpec((_NUM_CODES, _DIM), lambda i: (0, 0)),
        ],
        out_specs=[
            pl.BlockSpec((_R, _NUM_CODES), lambda i: (i, 0)),
            pl.BlockSpec((_R, _DIM), lambda i: (i, 0)),
            pl.BlockSpec((1, 1), lambda i: (0, 0)),
            pl.BlockSpec((1, 1), lambda i: (0, 0)),
        ],
        out_shape=[
            jax.ShapeDtypeStruct((_ROWS, _NUM_CODES), jnp.float32),
            jax.ShapeDtypeStruct((_ROWS, _DIM), jnp.float32),
            jax.ShapeDtypeStruct((1, 1), jnp.float32),
            jax.ShapeDtypeStruct((1, 1), jnp.float32),
        ],
        scratch_shapes=[
            pltpu.VMEM((1, _NUM_CODES), jnp.float32),
            pltpu.VMEM((1, _NUM_CODES), jnp.float32),
            pltpu.SMEM((1,), jnp.float32),
        ],
    )(x, embt, embedding_weight)
    quantized_out = jnp.transpose(q.reshape(b, h, w, c), (0, 3, 1, 2))
    encodings_out = enc.reshape(b, h, w, _NUM_CODES)
    return quantized_out, loss.reshape(()), perp.reshape(()), encodings_out
